# baseline (speedup 1.0000x reference)
import functools

import jax
import jax.numpy as jnp
from jax import lax
from jax.experimental import pallas as pl
from jax.experimental.pallas import tpu as pltpu
from jax.experimental.pallas import tpu_sc as plsc

B, S, D, E, K, OUT = 4, 4096, 1024, 8, 2, 1024
T = B * S
NPAIR = (E * (E - 1)) // 2
M = 128
PT = T + NPAIR * M
NT = PT // M
NTP = (NT + 15) // 16 * 16

NW2 = 16
TPW2 = T // NW2
NW = 32
RPW3 = PT // NW
TPW5 = T // NW

_OCC0 = 1


def _pair_start(a: int) -> int:
    return 7 * a - (a * (a - 1)) // 2


def _gate_body(x_ref, wg_ref, bg_ref, pid_ref, wa_ref, wb_ref):
    x = x_ref[...]
    logits = jnp.dot(x, wg_ref[...], preferred_element_type=jnp.float32)
    logits = logits + bg_ref[...]
    n = logits.shape[0]
    iota = lax.broadcasted_iota(jnp.int32, (n, E), 1)
    m1 = jnp.max(logits, axis=1, keepdims=True)
    i1 = jnp.min(jnp.where(logits == m1, iota, E), axis=1, keepdims=True)
    masked = jnp.where(iota == i1, jnp.finfo(jnp.float32).min, logits)
    m2 = jnp.max(masked, axis=1, keepdims=True)
    i2 = jnp.min(jnp.where(masked == m2, iota, E), axis=1, keepdims=True)
    w1 = 1.0 / (1.0 + jnp.exp(m2 - m1))
    w2 = 1.0 - w1
    a = jnp.minimum(i1, i2)
    b = jnp.maximum(i1, i2)
    first_is_a = i1 < i2
    wa = jnp.where(first_is_a, w1, w2)
    wb = jnp.where(first_is_a, w2, w1)
    pid = 7 * a - (a * (a - 1)) // 2 + (b - a - 1)
    pid_ref[...] = pid
    wa_ref[...] = wa
    wb_ref[...] = wb


_TT1 = 1024


def _gate(x2d, wg, bg2d):
    return pl.pallas_call(
        _gate_body,
        grid=(T // _TT1,),
        in_specs=[
            pl.BlockSpec((_TT1, D), lambda t: (t, 0)),
            pl.BlockSpec((D, E), lambda t: (0, 0)),
            pl.BlockSpec((1, E), lambda t: (0, 0)),
        ],
        out_specs=[
            pl.BlockSpec((_TT1, 1), lambda t: (t, 0)),
            pl.BlockSpec((_TT1, 1), lambda t: (t, 0)),
            pl.BlockSpec((_TT1, 1), lambda t: (t, 0)),
        ],
        out_shape=[
            jax.ShapeDtypeStruct((T, 1), jnp.int32),
            jax.ShapeDtypeStruct((T, 1), jnp.float32),
            jax.ShapeDtypeStruct((T, 1), jnp.float32),
        ],
    )(x2d, wg, bg2d)


_mesh1 = plsc.VectorSubcoreMesh(core_axis_name="c", subcore_axis_name="s",
                                num_cores=1)


@functools.partial(
    pl.kernel,
    out_type=[
        jax.ShapeDtypeStruct((PT,), jnp.int32),
        jax.ShapeDtypeStruct((PT,), jnp.float32),
        jax.ShapeDtypeStruct((PT,), jnp.float32),
        jax.ShapeDtypeStruct((NTP,), jnp.int32),
        jax.ShapeDtypeStruct((NTP,), jnp.int32),
        jax.ShapeDtypeStruct((T,), jnp.int32),
    ],
    mesh=_mesh1,
    compiler_params=pltpu.CompilerParams(needs_layout_passes=False),
    scratch_types=[
        pltpu.VMEM((TPW2,), jnp.int32),
        pltpu.VMEM((TPW2,), jnp.float32),
        pltpu.VMEM((TPW2,), jnp.float32),
        pltpu.VMEM((TPW2,), jnp.int32),
        pltpu.VMEM((TPW2,), jnp.int32),
        pltpu.VMEM((32,), jnp.int32),
        pltpu.VMEM((NW2 * 32,), jnp.int32),
        pltpu.VMEM((32,), jnp.int32),
        pltpu.VMEM((32,), jnp.int32),
        pltpu.VMEM((NTP,), jnp.int32),
        pltpu.VMEM((NTP,), jnp.int32),
        pltpu.VMEM_SHARED((NW2 * 32,), jnp.int32),
        pltpu.VMEM_SHARED((PT,), jnp.int32),
        pltpu.VMEM_SHARED((PT,), jnp.float32),
        pltpu.VMEM_SHARED((PT,), jnp.float32),
        pltpu.SemaphoreType.DMA,
    ],
)
def _route(pid_hbm, wa_hbm, wb_hbm, st_hbm, was_hbm, wbs_hbm, ta_hbm, tb_hbm,
           pos_hbm, pid_v, wa_v, wb_v, tok_v, rank_v, counts_v, call_v,
           base_v, cp_v, ta_v, tb_v, csh, st_sh, was_sh, wbs_sh, sem):
    wid = lax.axis_index("s")
    base = wid * TPW2
    nv = TPW2 // 16
    zeros16 = jnp.zeros((16,), jnp.int32)
    iota16 = lax.iota(jnp.int32, 16)

    zpw = PT // NW2
    ibase = wid * zpw
    for v in range(TPW2 // 16):
        fill = ibase + v * 16 + iota16
        fill = jnp.where(fill >= T, fill - T, fill)
        rank_v[pl.ds(v * 16, 16)] = fill
        wa_v[pl.ds(v * 16, 16)] = jnp.zeros((16,), jnp.float32)
    pltpu.sync_copy(rank_v, st_sh.at[pl.ds(ibase, TPW2)])
    pltpu.sync_copy(rank_v.at[pl.ds(0, zpw - TPW2)],
                    st_sh.at[pl.ds(ibase + TPW2, zpw - TPW2)])
    pltpu.sync_copy(wa_v, was_sh.at[pl.ds(ibase, TPW2)])
    pltpu.sync_copy(wa_v.at[pl.ds(0, zpw - TPW2)],
                    was_sh.at[pl.ds(ibase + TPW2, zpw - TPW2)])
    pltpu.sync_copy(wa_v, wbs_sh.at[pl.ds(ibase, TPW2)])
    pltpu.sync_copy(wa_v.at[pl.ds(0, zpw - TPW2)],
                    wbs_sh.at[pl.ds(ibase + TPW2, zpw - TPW2)])

    pltpu.sync_copy(pid_hbm.at[pl.ds(base, TPW2)], pid_v)
    pltpu.sync_copy(wa_hbm.at[pl.ds(base, TPW2)], wa_v)
    pltpu.sync_copy(wb_hbm.at[pl.ds(base, TPW2)], wb_v)

    counts_v[pl.ds(0, 16)] = zeros16
    counts_v[pl.ds(16, 16)] = zeros16
    for v in range(nv):
        pidv = pid_v[pl.ds(v * 16, 16)]
        occ, last = plsc.scan_count(pidv)
        cnt = occ - _OCC0 + 1
        plsc.addupdate_scatter(counts_v, [pidv], cnt, mask=last)
    pltpu.sync_copy(counts_v, csh.at[pl.ds(wid * 32, 32)])
    plsc.subcore_barrier()
    pltpu.sync_copy(csh, call_v)

    tot1 = zeros16
    tot2 = zeros16
    base0_1 = zeros16
    base0_2 = zeros16
    for w in range(NW2):
        row1 = call_v[pl.ds(w * 32, 16)]
        row2 = call_v[pl.ds(w * 32 + 16, 16)]
        mine = jnp.where(jnp.full((16,), w, jnp.int32) < wid, 1, 0)
        base0_1 = base0_1 + row1 * mine
        base0_2 = base0_2 + row2 * mine
        tot1 = tot1 + row1
        tot2 = tot2 + row2
    pad1 = jnp.bitwise_and(tot1 + (M - 1), jnp.full((16,), ~(M - 1), jnp.int32))
    pad2 = jnp.bitwise_and(tot2 + (M - 1), jnp.full((16,), ~(M - 1), jnp.int32))
    c1 = plsc.cumsum(pad1)
    excl1 = c1 - pad1
    tot_first = jnp.sum(pad1)
    excl2 = plsc.cumsum(pad2) - pad2 + tot_first
    cp_v[pl.ds(0, 16)] = excl1
    cp_v[pl.ds(16, 16)] = excl2
    base_v[pl.ds(0, 16)] = excl1 + base0_1
    base_v[pl.ds(16, 16)] = excl2 + base0_2

    for v in range(nv):
        pidv = pid_v[pl.ds(v * 16, 16)]
        occ, last = plsc.scan_count(pidv)
        basev = plsc.load_gather(base_v, [pidv])
        rank_v[pl.ds(v * 16, 16)] = basev + (occ - _OCC0)
        plsc.addupdate_scatter(base_v, [pidv], occ - _OCC0 + 1, mask=last)
        tok_v[pl.ds(v * 16, 16)] = base + v * 16 + iota16
    pltpu.sync_copy(tok_v, st_sh.at[rank_v])
    pltpu.sync_copy(wa_v, was_sh.at[rank_v])
    pltpu.sync_copy(wb_v, wbs_sh.at[rank_v])
    pltpu.sync_copy(rank_v, pos_hbm.at[pl.ds(base, TPW2)])
    plsc.subcore_barrier()
    for off, sz in ((0, TPW2), (TPW2, zpw - TPW2)):
        pltpu.sync_copy(st_sh.at[pl.ds(ibase + off, sz)],
                        rank_v.at[pl.ds(0, sz)])
        pltpu.sync_copy(rank_v.at[pl.ds(0, sz)],
                        st_hbm.at[pl.ds(ibase + off, sz)])
        pltpu.sync_copy(was_sh.at[pl.ds(ibase + off, sz)],
                        wa_v.at[pl.ds(0, sz)])
        pltpu.sync_copy(wa_v.at[pl.ds(0, sz)],
                        was_hbm.at[pl.ds(ibase + off, sz)])
        pltpu.sync_copy(wbs_sh.at[pl.ds(ibase + off, sz)],
                        wb_v.at[pl.ds(0, sz)])
        pltpu.sync_copy(wb_v.at[pl.ds(0, sz)],
                        wbs_hbm.at[pl.ds(ibase + off, sz)])

    @pl.when(wid == 0)
    def _tables():
        for tv in range(NTP // 16):
            t_ids = lax.iota(jnp.int32, 16) + tv * 16
            row0 = t_ids * M
            binv = jnp.zeros((16,), jnp.int32)
            for g in range(1, NPAIR):
                cpg = plsc.load_gather(cp_v, [jnp.full((16,), g, jnp.int32)])
                binv = binv + jnp.where(row0 >= cpg, 1, 0)
            a_t = jnp.zeros((16,), jnp.int32)
            for av in range(E - 1):
                s_next = _pair_start(av + 1)
                a_t = a_t + jnp.where(binv >= s_next, 1, 0)
            sa = 7 * a_t - lax.shift_right_logical(a_t * (a_t - 1), 1)
            b_t = binv - sa + a_t + 1
            ta_v[pl.ds(tv * 16, 16)] = a_t
            tb_v[pl.ds(tv * 16, 16)] = b_t
        pltpu.sync_copy(ta_v, ta_hbm)
        pltpu.sync_copy(tb_v, tb_hbm)


_mesh2 = plsc.VectorSubcoreMesh(core_axis_name="c", subcore_axis_name="s")


@functools.partial(
    pl.kernel,
    out_type=jax.ShapeDtypeStruct((PT, D), jnp.float32),
    mesh=_mesh2,
    scratch_types=[
        pltpu.VMEM((RPW3,), jnp.int32),
        pltpu.VMEM((64, D), jnp.float32),
        pltpu.SemaphoreType.DMA,
    ],
)
def _gather_rows(x_hbm, st_hbm, xs_hbm, idx_v, buf_v, sem):
    wid = lax.axis_index("s") * 2 + lax.axis_index("c")
    base = wid * RPW3
    pltpu.sync_copy(st_hbm.at[pl.ds(base, RPW3)], idx_v)
    offs = list(range(0, RPW3 - 64, 64)) + [RPW3 - 64]
    for off in offs:
        pltpu.async_copy(x_hbm.at[idx_v.at[pl.ds(off, 64)]], buf_v, sem).wait()
        pltpu.sync_copy(buf_v, xs_hbm.at[pl.ds(base + off, 64)])


def _pmm_body(xs_ref, wa_ref, wb_ref, we_ref, be_ref, ta_ref, tb_ref,
              out_ref):
    t = pl.program_id(0)
    a = ta_ref[t]
    b = tb_ref[t]
    x = xs_ref[...]
    wea = we_ref[a]
    web = we_ref[b]
    ya = jnp.dot(x, wea, preferred_element_type=jnp.float32)
    ya = ya + be_ref[pl.ds(a, 1), :]
    yb = jnp.dot(x, web, preferred_element_type=jnp.float32)
    yb = yb + be_ref[pl.ds(b, 1), :]
    out_ref[...] = ya * wa_ref[...] + yb * wb_ref[...]


def _pair_matmul(xs, was2, wbs2, we, be, ta, tb):
    return pl.pallas_call(
        _pmm_body,
        grid=(NT,),
        in_specs=[
            pl.BlockSpec((M, D), lambda t: (t, 0)),
            pl.BlockSpec((M, 1), lambda t: (t, 0)),
            pl.BlockSpec((M, 1), lambda t: (t, 0)),
            pl.BlockSpec((E, D, OUT), lambda t: (0, 0, 0)),
            pl.BlockSpec((E, OUT), lambda t: (0, 0)),
            pl.BlockSpec(memory_space=pltpu.SMEM),
            pl.BlockSpec(memory_space=pltpu.SMEM),
        ],
        out_specs=pl.BlockSpec((M, OUT), lambda t: (t, 0)),
        out_shape=jax.ShapeDtypeStruct((PT, OUT), jnp.float32),
    )(xs, was2, wbs2, we, be, ta, tb)


@functools.partial(
    pl.kernel,
    out_type=jax.ShapeDtypeStruct((T, OUT), jnp.float32),
    mesh=_mesh2,
    scratch_types=[
        pltpu.VMEM((TPW5,), jnp.int32),
        pltpu.VMEM((64, OUT), jnp.float32),
        pltpu.SemaphoreType.DMA,
    ],
)
def _unpermute(y_hbm, pos_hbm, out_hbm, pos_v, buf_v, sem):
    wid = lax.axis_index("s") * 2 + lax.axis_index("c")
    base = wid * TPW5
    pltpu.sync_copy(pos_hbm.at[pl.ds(base, TPW5)], pos_v)
    for c in range(TPW5 // 64):
        off = c * 64
        pltpu.async_copy(y_hbm.at[pos_v.at[pl.ds(off, 64)]], buf_v, sem).wait()
        pltpu.sync_copy(buf_v, out_hbm.at[pl.ds(base + off, 64)])


def kernel(inputs, Wg, bg, We, be):
    x2d = inputs.reshape(T, D)
    pid, wa, wb = _gate(x2d, Wg, bg.reshape(1, E))
    st, was, wbs, ta, tb, pos = _route(
        pid.reshape(T), wa.reshape(T), wb.reshape(T))
    xs = _gather_rows(x2d, st)
    yp = _pair_matmul(xs, was.reshape(PT, 1), wbs.reshape(PT, 1), We, be,
                      ta, tb)
    res = _unpermute(yp, pos)
    return res.reshape(B, S, OUT)


# speedup vs baseline: 1.0500x; 1.0500x over previous
import functools

import jax
import jax.numpy as jnp
from jax import lax
from jax.experimental import pallas as pl
from jax.experimental.pallas import tpu as pltpu
from jax.experimental.pallas import tpu_sc as plsc

B, S, D, E, K, OUT = 4, 4096, 1024, 8, 2, 1024
T = B * S
NPAIR = (E * (E - 1)) // 2
M = 256
PT = T + NPAIR * M
NT = PT // M
NTP = (NT + 15) // 16 * 16

NW2 = 16
TPW2 = T // NW2
NW = 32
RPW3 = PT // NW
TPW5 = T // NW

_OCC0 = 1


def _pair_start(a: int) -> int:
    return 7 * a - (a * (a - 1)) // 2


def _gate_body(x_ref, wg_ref, bg_ref, pid_ref, wa_ref, wb_ref):
    x = x_ref[...]
    logits = jnp.dot(x, wg_ref[...], preferred_element_type=jnp.float32)
    logits = logits + bg_ref[...]
    n = logits.shape[0]
    iota = lax.broadcasted_iota(jnp.int32, (n, E), 1)
    m1 = jnp.max(logits, axis=1, keepdims=True)
    i1 = jnp.min(jnp.where(logits == m1, iota, E), axis=1, keepdims=True)
    masked = jnp.where(iota == i1, jnp.finfo(jnp.float32).min, logits)
    m2 = jnp.max(masked, axis=1, keepdims=True)
    i2 = jnp.min(jnp.where(masked == m2, iota, E), axis=1, keepdims=True)
    w1 = 1.0 / (1.0 + jnp.exp(m2 - m1))
    w2 = 1.0 - w1
    a = jnp.minimum(i1, i2)
    b = jnp.maximum(i1, i2)
    first_is_a = i1 < i2
    wa = jnp.where(first_is_a, w1, w2)
    wb = jnp.where(first_is_a, w2, w1)
    pid = 7 * a - (a * (a - 1)) // 2 + (b - a - 1)
    pid_ref[...] = pid
    wa_ref[...] = wa
    wb_ref[...] = wb


_TT1 = 1024


def _gate(x2d, wg, bg2d):
    return pl.pallas_call(
        _gate_body,
        grid=(T // _TT1,),
        in_specs=[
            pl.BlockSpec((_TT1, D), lambda t: (t, 0)),
            pl.BlockSpec((D, E), lambda t: (0, 0)),
            pl.BlockSpec((1, E), lambda t: (0, 0)),
        ],
        out_specs=[
            pl.BlockSpec((_TT1, 1), lambda t: (t, 0)),
            pl.BlockSpec((_TT1, 1), lambda t: (t, 0)),
            pl.BlockSpec((_TT1, 1), lambda t: (t, 0)),
        ],
        out_shape=[
            jax.ShapeDtypeStruct((T, 1), jnp.int32),
            jax.ShapeDtypeStruct((T, 1), jnp.float32),
            jax.ShapeDtypeStruct((T, 1), jnp.float32),
        ],
    )(x2d, wg, bg2d)


_mesh1 = plsc.VectorSubcoreMesh(core_axis_name="c", subcore_axis_name="s",
                                num_cores=1)


@functools.partial(
    pl.kernel,
    out_type=[
        jax.ShapeDtypeStruct((PT,), jnp.int32),
        jax.ShapeDtypeStruct((PT,), jnp.float32),
        jax.ShapeDtypeStruct((PT,), jnp.float32),
        jax.ShapeDtypeStruct((NTP,), jnp.int32),
        jax.ShapeDtypeStruct((NTP,), jnp.int32),
        jax.ShapeDtypeStruct((T,), jnp.int32),
    ],
    mesh=_mesh1,
    compiler_params=pltpu.CompilerParams(needs_layout_passes=False),
    scratch_types=[
        pltpu.VMEM((TPW2,), jnp.int32),
        pltpu.VMEM((TPW2,), jnp.float32),
        pltpu.VMEM((TPW2,), jnp.float32),
        pltpu.VMEM((TPW2,), jnp.int32),
        pltpu.VMEM((TPW2,), jnp.int32),
        pltpu.VMEM((32,), jnp.int32),
        pltpu.VMEM((NW2 * 32,), jnp.int32),
        pltpu.VMEM((32,), jnp.int32),
        pltpu.VMEM((32,), jnp.int32),
        pltpu.VMEM((NTP,), jnp.int32),
        pltpu.VMEM((NTP,), jnp.int32),
        pltpu.VMEM_SHARED((NW2 * 32,), jnp.int32),
        pltpu.VMEM_SHARED((PT,), jnp.int32),
        pltpu.VMEM_SHARED((PT,), jnp.float32),
        pltpu.VMEM_SHARED((PT,), jnp.float32),
        pltpu.SemaphoreType.DMA,
    ],
)
def _route(pid_hbm, wa_hbm, wb_hbm, st_hbm, was_hbm, wbs_hbm, ta_hbm, tb_hbm,
           pos_hbm, pid_v, wa_v, wb_v, tok_v, rank_v, counts_v, call_v,
           base_v, cp_v, ta_v, tb_v, csh, st_sh, was_sh, wbs_sh, sem):
    wid = lax.axis_index("s")
    base = wid * TPW2
    nv = TPW2 // 16
    zeros16 = jnp.zeros((16,), jnp.int32)
    iota16 = lax.iota(jnp.int32, 16)

    zpw = PT // NW2
    ibase = wid * zpw
    for v in range(TPW2 // 16):
        fill = ibase + v * 16 + iota16
        fill = jnp.where(fill >= T, fill - T, fill)
        rank_v[pl.ds(v * 16, 16)] = fill
        wa_v[pl.ds(v * 16, 16)] = jnp.zeros((16,), jnp.float32)
    pltpu.sync_copy(rank_v, st_sh.at[pl.ds(ibase, TPW2)])
    pltpu.sync_copy(rank_v.at[pl.ds(0, zpw - TPW2)],
                    st_sh.at[pl.ds(ibase + TPW2, zpw - TPW2)])
    pltpu.sync_copy(wa_v, was_sh.at[pl.ds(ibase, TPW2)])
    pltpu.sync_copy(wa_v.at[pl.ds(0, zpw - TPW2)],
                    was_sh.at[pl.ds(ibase + TPW2, zpw - TPW2)])
    pltpu.sync_copy(wa_v, wbs_sh.at[pl.ds(ibase, TPW2)])
    pltpu.sync_copy(wa_v.at[pl.ds(0, zpw - TPW2)],
                    wbs_sh.at[pl.ds(ibase + TPW2, zpw - TPW2)])

    pltpu.sync_copy(pid_hbm.at[pl.ds(base, TPW2)], pid_v)
    pltpu.sync_copy(wa_hbm.at[pl.ds(base, TPW2)], wa_v)
    pltpu.sync_copy(wb_hbm.at[pl.ds(base, TPW2)], wb_v)

    counts_v[pl.ds(0, 16)] = zeros16
    counts_v[pl.ds(16, 16)] = zeros16
    for v in range(nv):
        pidv = pid_v[pl.ds(v * 16, 16)]
        occ, last = plsc.scan_count(pidv)
        cnt = occ - _OCC0 + 1
        plsc.addupdate_scatter(counts_v, [pidv], cnt, mask=last)
    pltpu.sync_copy(counts_v, csh.at[pl.ds(wid * 32, 32)])
    plsc.subcore_barrier()
    pltpu.sync_copy(csh, call_v)

    tot1 = zeros16
    tot2 = zeros16
    base0_1 = zeros16
    base0_2 = zeros16
    for w in range(NW2):
        row1 = call_v[pl.ds(w * 32, 16)]
        row2 = call_v[pl.ds(w * 32 + 16, 16)]
        mine = jnp.where(jnp.full((16,), w, jnp.int32) < wid, 1, 0)
        base0_1 = base0_1 + row1 * mine
        base0_2 = base0_2 + row2 * mine
        tot1 = tot1 + row1
        tot2 = tot2 + row2
    pad1 = jnp.bitwise_and(tot1 + (M - 1), jnp.full((16,), ~(M - 1), jnp.int32))
    pad2 = jnp.bitwise_and(tot2 + (M - 1), jnp.full((16,), ~(M - 1), jnp.int32))
    c1 = plsc.cumsum(pad1)
    excl1 = c1 - pad1
    tot_first = jnp.sum(pad1)
    excl2 = plsc.cumsum(pad2) - pad2 + tot_first
    cp_v[pl.ds(0, 16)] = excl1
    cp_v[pl.ds(16, 16)] = excl2
    base_v[pl.ds(0, 16)] = excl1 + base0_1
    base_v[pl.ds(16, 16)] = excl2 + base0_2

    for v in range(nv):
        pidv = pid_v[pl.ds(v * 16, 16)]
        occ, last = plsc.scan_count(pidv)
        basev = plsc.load_gather(base_v, [pidv])
        rank_v[pl.ds(v * 16, 16)] = basev + (occ - _OCC0)
        plsc.addupdate_scatter(base_v, [pidv], occ - _OCC0 + 1, mask=last)
        tok_v[pl.ds(v * 16, 16)] = base + v * 16 + iota16
    pltpu.sync_copy(tok_v, st_sh.at[rank_v])
    pltpu.sync_copy(wa_v, was_sh.at[rank_v])
    pltpu.sync_copy(wb_v, wbs_sh.at[rank_v])
    pltpu.sync_copy(rank_v, pos_hbm.at[pl.ds(base, TPW2)])
    plsc.subcore_barrier()
    for off, sz in ((0, TPW2), (TPW2, zpw - TPW2)):
        pltpu.sync_copy(st_sh.at[pl.ds(ibase + off, sz)],
                        rank_v.at[pl.ds(0, sz)])
        pltpu.sync_copy(rank_v.at[pl.ds(0, sz)],
                        st_hbm.at[pl.ds(ibase + off, sz)])
        pltpu.sync_copy(was_sh.at[pl.ds(ibase + off, sz)],
                        wa_v.at[pl.ds(0, sz)])
        pltpu.sync_copy(wa_v.at[pl.ds(0, sz)],
                        was_hbm.at[pl.ds(ibase + off, sz)])
        pltpu.sync_copy(wbs_sh.at[pl.ds(ibase + off, sz)],
                        wb_v.at[pl.ds(0, sz)])
        pltpu.sync_copy(wb_v.at[pl.ds(0, sz)],
                        wbs_hbm.at[pl.ds(ibase + off, sz)])

    @pl.when(wid == 0)
    def _tables():
        for tv in range(NTP // 16):
            t_ids = lax.iota(jnp.int32, 16) + tv * 16
            row0 = t_ids * M
            binv = jnp.zeros((16,), jnp.int32)
            for g in range(1, NPAIR):
                cpg = plsc.load_gather(cp_v, [jnp.full((16,), g, jnp.int32)])
                binv = binv + jnp.where(row0 >= cpg, 1, 0)
            a_t = jnp.zeros((16,), jnp.int32)
            for av in range(E - 1):
                s_next = _pair_start(av + 1)
                a_t = a_t + jnp.where(binv >= s_next, 1, 0)
            sa = 7 * a_t - lax.shift_right_logical(a_t * (a_t - 1), 1)
            b_t = binv - sa + a_t + 1
            ta_v[pl.ds(tv * 16, 16)] = a_t
            tb_v[pl.ds(tv * 16, 16)] = b_t
        pltpu.sync_copy(ta_v, ta_hbm)
        pltpu.sync_copy(tb_v, tb_hbm)


_mesh2 = plsc.VectorSubcoreMesh(core_axis_name="c", subcore_axis_name="s")


@functools.partial(
    pl.kernel,
    out_type=jax.ShapeDtypeStruct((PT, D), jnp.float32),
    mesh=_mesh2,
    scratch_types=[
        pltpu.VMEM((RPW3,), jnp.int32),
        pltpu.VMEM((48, D), jnp.float32),
        pltpu.VMEM((48, D), jnp.float32),
        pltpu.SemaphoreType.DMA,
        pltpu.SemaphoreType.DMA,
    ],
)
def _gather_rows(x_hbm, st_hbm, xs_hbm, idx_v, buf0, buf1, sem0, sem1):
    wid = lax.axis_index("s") * 2 + lax.axis_index("c")
    base = wid * RPW3
    pltpu.sync_copy(st_hbm.at[pl.ds(base, RPW3)], idx_v)
    offs = list(range(0, RPW3 - 48, 48)) + [RPW3 - 48]
    bufs = (buf0, buf1)
    sems = (sem0, sem1)
    handles = [None, None]
    for c, off in enumerate(offs):
        s = c % 2
        handles[s] = pltpu.async_copy(
            x_hbm.at[idx_v.at[pl.ds(off, 48)]], bufs[s], sems[s])
        if c >= 1:
            ps = (c - 1) % 2
            handles[ps].wait()
            pltpu.sync_copy(bufs[ps],
                            xs_hbm.at[pl.ds(base + offs[c - 1], 48)])
    lc = len(offs) - 1
    handles[lc % 2].wait()
    pltpu.sync_copy(bufs[lc % 2], xs_hbm.at[pl.ds(base + offs[lc], 48)])


def _pmm_body(xs_ref, wa_ref, wb_ref, we_ref, be_ref, ta_ref, tb_ref,
              out_ref):
    t = pl.program_id(0)
    a = ta_ref[t]
    b = tb_ref[t]
    x = xs_ref[...]
    wea = we_ref[a]
    web = we_ref[b]
    ya = jnp.dot(x, wea, preferred_element_type=jnp.float32)
    ya = ya + be_ref[pl.ds(a, 1), :]
    yb = jnp.dot(x, web, preferred_element_type=jnp.float32)
    yb = yb + be_ref[pl.ds(b, 1), :]
    out_ref[...] = ya * wa_ref[...] + yb * wb_ref[...]


def _pair_matmul(xs, was2, wbs2, we, be, ta, tb):
    return pl.pallas_call(
        _pmm_body,
        grid=(NT,),
        in_specs=[
            pl.BlockSpec((M, D), lambda t: (t, 0)),
            pl.BlockSpec((M, 1), lambda t: (t, 0)),
            pl.BlockSpec((M, 1), lambda t: (t, 0)),
            pl.BlockSpec((E, D, OUT), lambda t: (0, 0, 0)),
            pl.BlockSpec((E, OUT), lambda t: (0, 0)),
            pl.BlockSpec(memory_space=pltpu.SMEM),
            pl.BlockSpec(memory_space=pltpu.SMEM),
        ],
        out_specs=pl.BlockSpec((M, OUT), lambda t: (t, 0)),
        out_shape=jax.ShapeDtypeStruct((PT, OUT), jnp.float32),
    )(xs, was2, wbs2, we, be, ta, tb)


@functools.partial(
    pl.kernel,
    out_type=jax.ShapeDtypeStruct((T, OUT), jnp.float32),
    mesh=_mesh2,
    scratch_types=[
        pltpu.VMEM((TPW5,), jnp.int32),
        pltpu.VMEM((48, OUT), jnp.float32),
        pltpu.VMEM((48, OUT), jnp.float32),
        pltpu.SemaphoreType.DMA,
        pltpu.SemaphoreType.DMA,
    ],
)
def _unpermute(y_hbm, pos_hbm, out_hbm, pos_v, buf0, buf1, sem0, sem1):
    wid = lax.axis_index("s") * 2 + lax.axis_index("c")
    base = wid * TPW5
    pltpu.sync_copy(pos_hbm.at[pl.ds(base, TPW5)], pos_v)
    offs5 = list(range(0, TPW5 - 48, 48)) + [TPW5 - 48]
    nch = len(offs5)
    bufs = (buf0, buf1)
    sems = (sem0, sem1)
    handles = [None, None]
    for c in range(nch):
        s = c % 2
        handles[s] = pltpu.async_copy(
            y_hbm.at[pos_v.at[pl.ds(offs5[c], 48)]], bufs[s], sems[s])
        if c >= 1:
            ps = (c - 1) % 2
            handles[ps].wait()
            pltpu.sync_copy(bufs[ps],
                            out_hbm.at[pl.ds(base + offs5[c - 1], 48)])
    handles[(nch - 1) % 2].wait()
    pltpu.sync_copy(bufs[(nch - 1) % 2],
                    out_hbm.at[pl.ds(base + offs5[nch - 1], 48)])


def kernel(inputs, Wg, bg, We, be):
    x2d = inputs.reshape(T, D)
    pid, wa, wb = _gate(x2d, Wg, bg.reshape(1, E))
    st, was, wbs, ta, tb, pos = _route(
        pid.reshape(T), wa.reshape(T), wb.reshape(T))
    xs = _gather_rows(x2d, st)
    yp = _pair_matmul(xs, was.reshape(PT, 1), wbs.reshape(PT, 1), We, be,
                      ta, tb)
    res = _unpermute(yp, pos)
    return res.reshape(B, S, OUT)
